# baseline (speedup 1.0000x reference)
import functools
import numpy as np
import jax
import jax.numpy as jnp
from jax import lax
from jax.experimental import pallas as pl
from jax.experimental.pallas import tpu as pltpu, tpu_sc as plsc


def _build_idx_list():
    num_candidates = 16
    indices = [0, 1, 2, 3, 4, 5, 6, 7, 8]
    base_idx = 9
    for i in range(num_candidates - 1):
        indices += [6, 7, base_idx + i]
    indices += [0, 3, 6, 1, 4, 7, 2, 5, 8]
    for i in range(num_candidates - 1):
        indices += [2, 5, base_idx + i]
    return indices


_IDX = np.array(_build_idx_list(), dtype=np.int32)

_B, _N, _S, _D = 8, 24, 512, 128
_CPS = 8
_ROWW = (_S // _CPS) * _D
_NROWS = _B * 108 * _CPS
_NW = 32
_RPW = _NROWS // _NW
_K = 4
_NIT = _RPW // _K
_NBUF = 3
_T = _NIT // _NBUF


def _src_rows():
    b = np.arange(_B, dtype=np.int32)
    c = np.arange(_CPS, dtype=np.int32)
    src = (b[:, None, None] * _N + _IDX[None, :, None]) * _CPS + c[None, None, :]
    return src.reshape(_NW, _NIT, _K)


_SRC = _src_rows()


def kernel(x):
    x_flat = x.reshape(_B * _N * _CPS, _ROWW)
    src = jnp.asarray(_SRC)
    mesh = plsc.VectorSubcoreMesh(core_axis_name="c", subcore_axis_name="s")

    @functools.partial(
        pl.kernel,
        mesh=mesh,
        out_type=jax.ShapeDtypeStruct((_NROWS, _ROWW), jnp.float32),
        scratch_types=[
            pltpu.VMEM((_NIT, _K), jnp.int32),
            pltpu.VMEM((_K, _ROWW), jnp.float32),
            pltpu.VMEM((_K, _ROWW), jnp.float32),
            pltpu.VMEM((_K, _ROWW), jnp.float32),
            pltpu.SemaphoreType.DMA,
            pltpu.SemaphoreType.DMA,
            pltpu.SemaphoreType.DMA,
            pltpu.SemaphoreType.DMA,
            pltpu.SemaphoreType.DMA,
            pltpu.SemaphoreType.DMA,
        ],
    )
    def k(x_hbm, src_hbm, out_hbm, idx_v, b0, b1, b2, si0, si1, si2, so0, so1, so2):
        wid = lax.axis_index("s") * 2 + lax.axis_index("c")
        pltpu.sync_copy(src_hbm.at[wid], idx_v)
        base = wid * _RPW
        bufs = (b0, b1, b2)
        sis = (si0, si1, si2)
        sos = (so0, so1, so2)

        gathers = [None, None, None]
        for u in range(_NBUF):
            gathers[u] = pltpu.async_copy(x_hbm.at[idx_v.at[u]], bufs[u], sis[u])

        def step(t, carry):
            g = t * _NBUF
            nxt = jnp.minimum(g + _NBUF, _NIT - _NBUF)
            for u in range(_NBUF):
                pltpu.make_async_copy(x_hbm.at[idx_v.at[g + u]], bufs[u], sis[u]).wait()
                pltpu.async_copy(bufs[u], out_hbm.at[pl.ds(base + (g + u) * _K, _K)], sos[u])
            for u in range(_NBUF):
                pltpu.make_async_copy(bufs[u], out_hbm.at[pl.ds(base + (g + u) * _K, _K)], sos[u]).wait()
                pltpu.async_copy(x_hbm.at[idx_v.at[nxt + u]], bufs[u], sis[u])
            return carry

        lax.fori_loop(0, _T, step, 0)

        for u in range(_NBUF):
            pltpu.make_async_copy(x_hbm.at[idx_v.at[u]], bufs[u], sis[u]).wait()

    out = k(x_flat, src)
    return out.reshape(_B, 36, 3, _S, _D)


# speedup vs baseline: 5.7658x; 5.7658x over previous
import jax
import jax.numpy as jnp
from jax.experimental import pallas as pl
from jax.experimental.pallas import tpu as pltpu


def _build_idx_list():
    num_candidates = 16
    indices = [0, 1, 2, 3, 4, 5, 6, 7, 8]
    base_idx = 9
    for i in range(num_candidates - 1):
        indices += [6, 7, base_idx + i]
    indices += [0, 3, 6, 1, 4, 7, 2, 5, 8]
    for i in range(num_candidates - 1):
        indices += [2, 5, base_idx + i]
    return indices


_IDX = _build_idx_list()


def _merge_runs(idx):
    runs = []
    o_start, s_start, length = 0, idx[0], 1
    for j in range(1, len(idx)):
        if idx[j] == s_start + length:
            length += 1
        else:
            runs.append((o_start, s_start, length))
            o_start, s_start, length = j, idx[j], 1
    runs.append((o_start, s_start, length))
    return runs


_RUNS = _merge_runs(_IDX)


def _body(x_ref, o_hbm, sem):
    b = pl.program_id(0)
    copies = [
        pltpu.make_async_copy(
            x_ref.at[0, pl.ds(s_start, length)],
            o_hbm.at[b, pl.ds(o_start, length)],
            sem,
        )
        for o_start, s_start, length in _RUNS
    ]
    for c in copies:
        c.start()
    for c in copies:
        c.wait()


def kernel(x):
    b, n, s, d = x.shape
    n_out = len(_IDX)

    out = pl.pallas_call(
        _body,
        grid=(b,),
        in_specs=[pl.BlockSpec((1, n, s, d), lambda i: (i, 0, 0, 0))],
        out_specs=pl.BlockSpec(memory_space=pl.ANY),
        out_shape=jax.ShapeDtypeStruct((b, n_out, s, d), x.dtype),
        scratch_shapes=[pltpu.SemaphoreType.DMA],
    )(x)
    return out.reshape(b, n_out // 3, 3, s, d)
